# baseline (device time: 22829 ns/iter reference)
import jax
import jax.numpy as jnp
from jax import lax
from jax.experimental import pallas as pl
from jax.experimental.pallas import tpu as pltpu

N_CHUNKS = 4


def kernel(x, assign, W1, W2):
    T, D = x.shape
    E_loc, _, F = W1.shape

    assign2 = assign

    n_sems = 1 + 2 * N_CHUNKS
    ck = T // N_CHUNKS

    def body(x_ref, a_ref, w1_ref, w2_ref, out_ref,
             xvm, asend, xsend, xbuf, abuf, sbuf, rbuf,
             load_sem, send_sems, recv_sems):
        my_x = lax.axis_index("x")
        my_y = lax.axis_index("y")
        my_z = lax.axis_index("z")
        partner = (my_x, my_y, 1 - my_z)

        barrier_sem = pltpu.get_barrier_semaphore()
        pl.semaphore_signal(
            barrier_sem, inc=1,
            device_id=partner, device_id_type=pl.DeviceIdType.MESH,
        )
        ld_x = pltpu.make_async_copy(x_ref, xvm, load_sem)
        ld_x.start()
        asend[...] = a_ref[...].reshape(1, T).astype(jnp.bfloat16)
        ld_x.wait()
        xsend[...] = xvm[...].astype(jnp.bfloat16)
        pl.semaphore_wait(barrier_sem, 1)

        rdma_a = pltpu.make_async_remote_copy(
            src_ref=asend, dst_ref=abuf,
            send_sem=send_sems.at[0], recv_sem=recv_sems.at[0],
            device_id=partner, device_id_type=pl.DeviceIdType.MESH,
        )
        rdma_a.start()
        rdma_x = []
        for k in range(N_CHUNKS):
            sl = pl.ds(k * ck, ck)
            r = pltpu.make_async_remote_copy(
                src_ref=xsend.at[sl],
                dst_ref=xbuf.at[sl],
                send_sem=send_sems.at[1 + k], recv_sem=recv_sems.at[1 + k],
                device_id=partner, device_id_type=pl.DeviceIdType.MESH,
            )
            r.start()
            rdma_x.append(r)

        e0 = 2 * my_z

        ident = (
            lax.broadcasted_iota(jnp.int32, (T, T), 0)
            == lax.broadcasted_iota(jnp.int32, (T, T), 1)
        ).astype(jnp.float32)

        def to_col(row):
            return lax.dot_general(
                ident, row.astype(jnp.float32), (((1,), (1,)), ((), ())),
                preferred_element_type=jnp.float32,
            )

        def ffn(xv, a_col):
            acc = jnp.zeros((xv.shape[0], D), jnp.float32)
            for l in range(E_loc):
                m = (a_col == (e0 + l).astype(jnp.float32)).astype(xv.dtype)
                h = jnp.maximum(
                    jnp.dot(xv * m, w1_ref[l],
                            preferred_element_type=jnp.float32),
                    0.0,
                )
                acc = acc + jnp.dot(h, w2_ref[l],
                                    preferred_element_type=jnp.float32)
            return acc

        a_col_local = to_col(asend[...])
        acc_local = ffn(xsend[...], a_col_local)

        rdma_a.wait_recv()
        a_col_partner = to_col(abuf[...])

        rdma_r = []
        for k in range(N_CHUNKS):
            sl = pl.ds(k * ck, ck)
            rdma_x[k].wait_recv()
            sbuf[sl, :] = ffn(
                xbuf[sl, :], a_col_partner[k * ck:(k + 1) * ck, :]
            ).astype(jnp.bfloat16)
            r = pltpu.make_async_remote_copy(
                src_ref=sbuf.at[sl],
                dst_ref=rbuf.at[sl],
                send_sem=send_sems.at[1 + N_CHUNKS + k],
                recv_sem=recv_sems.at[1 + N_CHUNKS + k],
                device_id=partner, device_id_type=pl.DeviceIdType.MESH,
            )
            r.start()
            rdma_r.append(r)

        for k, r in enumerate(rdma_r):
            r.wait_recv()
            sl = pl.ds(k * ck, ck)
            out_ref[sl, :] = (
                acc_local[k * ck:(k + 1) * ck, :]
                + rbuf[sl, :].astype(jnp.float32)
            )

        rdma_a.wait_send()
        for r in rdma_x:
            r.wait_send()
        for r in rdma_r:
            r.wait_send()

    return pl.pallas_call(
        body,
        out_shape=jax.ShapeDtypeStruct((T, D), jnp.float32),
        in_specs=[
            pl.BlockSpec(memory_space=pl.ANY),
            pl.BlockSpec(memory_space=pltpu.VMEM),
            pl.BlockSpec(memory_space=pltpu.VMEM),
            pl.BlockSpec(memory_space=pltpu.VMEM),
        ],
        out_specs=pl.BlockSpec(memory_space=pltpu.VMEM),
        scratch_shapes=[
            pltpu.VMEM((T, D), jnp.float32),
            pltpu.VMEM((1, T), jnp.bfloat16),
            pltpu.VMEM((T, D), jnp.bfloat16),
            pltpu.VMEM((T, D), jnp.bfloat16),
            pltpu.VMEM((1, T), jnp.bfloat16),
            pltpu.VMEM((T, D), jnp.bfloat16),
            pltpu.VMEM((T, D), jnp.bfloat16),
            pltpu.SemaphoreType.DMA,
            pltpu.SemaphoreType.DMA((n_sems,)),
            pltpu.SemaphoreType.DMA((n_sems,)),
        ],
        compiler_params=pltpu.CompilerParams(collective_id=0),
    )(x, assign2, W1, W2)
